# Initial kernel scaffold; baseline (speedup 1.0000x reference)
import functools

import jax
import jax.numpy as jnp
from jax import lax
from jax.experimental import pallas as pl
from jax.experimental.pallas import tpu as pltpu
from jax.experimental.pallas import tpu_sc as plsc

N = 215820
E = 647460
D = 96
OUT = 48

NC = 2
NS = 16
NW = NC * NS

CSH = 14
CHUNK = 1 << CSH
C = (N + CHUNK - 1) // CHUNK
NPAD = C * CHUNK

EPT = 20240
EPAD = EPT * NW

BLK = 4048
NV = BLK // 16
NBLKS = EPT // BLK

FB = 2048
FTH = 2016
CAP = 22400

K = 128

_mesh = functools.partial(
    plsc.VectorSubcoreMesh, core_axis_name="c", subcore_axis_name="s",
    num_cores=NC, num_subcores=NS)


def _wid():
    return lax.axis_index("s") * NC + lax.axis_index("c")


def _partition_body(srcp, dstp, bsrc, bdst, counts,
                    sbuf, dbuf, fsrc, fdst, cnt_v, ptrs, outc):
    wid = _wid()
    base = wid * EPT
    for c in range(C):
        ptrs[c] = jnp.int32(0)
        outc[c] = jnp.int32(0)

    for blk in range(NBLKS):
        gbase = base + blk * BLK
        pltpu.sync_copy(srcp.at[pl.ds(gbase, BLK)], sbuf)
        pltpu.sync_copy(dstp.at[pl.ds(gbase, BLK)], dbuf)

        def vbody(v, _, gbase=gbase):
            sv = sbuf[pl.ds(v * 16, 16)]
            dv = dbuf[pl.ds(v * 16, 16)]
            valid = (gbase + v * 16 + lax.iota(jnp.int32, 16)) < E
            ch = lax.shift_right_logical(dv, CSH)
            dl = jnp.bitwise_and(dv, CHUNK - 1)
            for c in range(C):
                m = jnp.logical_and(ch == c, valid)
                p = ptrs[c]
                plsc.store_compressed(fsrc.at[c, pl.ds(p, 16)], sv, mask=m)
                plsc.store_compressed(fdst.at[c, pl.ds(p, 16)], dl, mask=m)
                cnt = jnp.max(plsc.all_reduce_population_count(m))
                p2 = p + cnt
                ptrs[c] = p2

                @pl.when(p2 >= FTH)
                def _(c=c, p2=p2):
                    oc = outc[c]
                    pltpu.sync_copy(fsrc.at[c, pl.ds(0, FTH)],
                                    bsrc.at[wid, c, pl.ds(oc, FTH)])
                    pltpu.sync_copy(fdst.at[c, pl.ds(0, FTH)],
                                    bdst.at[wid, c, pl.ds(oc, FTH)])
                    outc[c] = oc + FTH
                    fsrc[c, pl.ds(0, 16)] = fsrc[c, pl.ds(FTH, 16)]
                    fdst[c, pl.ds(0, 16)] = fdst[c, pl.ds(FTH, 16)]
                    ptrs[c] = p2 - FTH
            return 0

        lax.fori_loop(0, NV, vbody, 0)

    lanes = lax.iota(jnp.int32, 16)
    cvals = jnp.zeros((16,), jnp.int32)
    for c in range(C):
        p = ptrs[c]
        fsrc[c, pl.ds(p, 16)] = jnp.zeros((16,), jnp.int32)
        fdst[c, pl.ds(p, 16)] = jnp.full((16,), CHUNK, jnp.int32)
        p16 = jnp.bitwise_and(p + 15, jnp.int32(~15))
        oc = outc[c]
        pltpu.sync_copy(fsrc.at[c, pl.ds(0, FTH)],
                        bsrc.at[wid, c, pl.ds(oc, FTH)])
        pltpu.sync_copy(fdst.at[c, pl.ds(0, FTH)],
                        bdst.at[wid, c, pl.ds(oc, FTH)])
        cvals = jnp.where(lanes == c, oc + p16, cvals)
    cnt_v[...] = cvals
    pltpu.sync_copy(cnt_v, counts.at[wid])


def _run_partition(srcp, dstp):
    f = pl.kernel(
        _partition_body,
        out_type=[
            jax.ShapeDtypeStruct((NW, C, CAP), jnp.int32),
            jax.ShapeDtypeStruct((NW, C, CAP), jnp.int32),
            jax.ShapeDtypeStruct((NW, 16), jnp.int32),
        ],
        mesh=_mesh(),
        scratch_types=[
            pltpu.VMEM((BLK,), jnp.int32),
            pltpu.VMEM((BLK,), jnp.int32),
            pltpu.VMEM((C, FB), jnp.int32),
            pltpu.VMEM((C, FB), jnp.int32),
            pltpu.VMEM((16,), jnp.int32),
            pltpu.SMEM((16,), jnp.int32),
            pltpu.SMEM((16,), jnp.int32),
        ],
    )
    return f(srcp, dstp)


def _scatter_body(y_hbm, bsrc, bdst, counts, zeros_hbm, S_hbm,
                  ids_s, ids_d, ids_s16, ids_d16, rows, rows16,
                  zbuf, counts_v, acc, sem):
    core = lax.axis_index("c")
    sid = lax.axis_index("s")
    wid = sid * NC + core
    tb = sid * (CHUNK // NS)
    pltpu.sync_copy(counts.at[wid], counts_v)
    pltpu.sync_copy(zeros_hbm, zbuf)

    for c in range(C):
        @pl.when(jnp.equal(core, c % NC))
        def _(c=c):
            for z in range(CHUNK // NS // 64):
                pltpu.sync_copy(zbuf, acc.at[pl.ds(tb + z * 64, 64)])
            plsc.subcore_barrier()

            cnt = counts_v[c]
            nblk = lax.shift_right_logical(cnt, 7)

            def blkbody(b, _):
                off = b * K
                pltpu.sync_copy(bsrc.at[wid, c, pl.ds(off, K)], ids_s)
                pltpu.sync_copy(bdst.at[wid, c, pl.ds(off, K)], ids_d)
                pltpu.async_copy(y_hbm.at[ids_s], rows, sem).wait()
                pltpu.sync_copy(rows, acc.at[ids_d], add=True)
                return 0

            lax.fori_loop(0, nblk, blkbody, 0)

            rem16 = jnp.bitwise_and(lax.shift_right_logical(cnt, 4), 7)
            tail0 = nblk * K

            def tbody(b, _):
                off = tail0 + b * 16
                pltpu.sync_copy(bsrc.at[wid, c, pl.ds(off, 16)], ids_s16)
                pltpu.sync_copy(bdst.at[wid, c, pl.ds(off, 16)], ids_d16)
                pltpu.async_copy(y_hbm.at[ids_s16], rows16, sem).wait()
                pltpu.sync_copy(rows16, acc.at[ids_d16], add=True)
                return 0

            lax.fori_loop(0, rem16, tbody, 0)
            plsc.subcore_barrier()
            pltpu.sync_copy(acc.at[pl.ds(tb, CHUNK // NS)],
                            S_hbm.at[pl.ds(c * CHUNK + tb, CHUNK // NS)])


def _make_scatter(width):
    return pl.kernel(
        _scatter_body,
        out_type=jax.ShapeDtypeStruct((NPAD, width), jnp.float32),
        mesh=_mesh(),
        scratch_types=[
            pltpu.VMEM((K,), jnp.int32),
            pltpu.VMEM((K,), jnp.int32),
            pltpu.VMEM((16,), jnp.int32),
            pltpu.VMEM((16,), jnp.int32),
            pltpu.VMEM((K, width), jnp.float32),
            pltpu.VMEM((16, width), jnp.float32),
            pltpu.VMEM((64, width), jnp.float32),
            pltpu.VMEM((16,), jnp.int32),
            pltpu.VMEM_SHARED((CHUNK + 8, width), jnp.float32),
            pltpu.SemaphoreType.DMA,
        ],
    )


def _run_scatter(y, bsrc, bdst, counts, width):
    zeros = jnp.zeros((64, width), jnp.float32)
    return _make_scatter(width)(y, bsrc, bdst, counts, zeros)


def _deg_body(bdst, counts, ones_hbm, zeros_hbm, deg_hbm,
              ids_d, ids_d16, onesv, ones16, zbuf, counts_v, acc, sem):
    core = lax.axis_index("c")
    sid = lax.axis_index("s")
    wid = sid * NC + core
    tb = sid * (CHUNK // NS)
    pltpu.sync_copy(counts.at[wid], counts_v)
    pltpu.sync_copy(ones_hbm, onesv)
    pltpu.sync_copy(ones_hbm.at[pl.ds(0, 16)], ones16)
    pltpu.sync_copy(zeros_hbm, zbuf)

    for c in range(C):
        @pl.when(jnp.equal(core, c % NC))
        def _(c=c):
            for z in range(CHUNK // NS // 64):
                pltpu.sync_copy(zbuf, acc.at[pl.ds(tb + z * 64, 64)])
            plsc.subcore_barrier()

            cnt = counts_v[c]
            nblk = lax.shift_right_logical(cnt, 7)

            def blkbody(b, _):
                off = b * K
                pltpu.sync_copy(bdst.at[wid, c, pl.ds(off, K)], ids_d)
                pltpu.sync_copy(onesv, acc.at[ids_d], add=True)
                return 0

            lax.fori_loop(0, nblk, blkbody, 0)

            rem16 = jnp.bitwise_and(lax.shift_right_logical(cnt, 4), 7)
            tail0 = nblk * K

            def tbody(b, _):
                off = tail0 + b * 16
                pltpu.sync_copy(bdst.at[wid, c, pl.ds(off, 16)], ids_d16)
                pltpu.sync_copy(ones16, acc.at[ids_d16], add=True)
                return 0

            lax.fori_loop(0, rem16, tbody, 0)
            plsc.subcore_barrier()
            pltpu.sync_copy(acc.at[pl.ds(tb, CHUNK // NS)],
                            deg_hbm.at[pl.ds(c * CHUNK + tb, CHUNK // NS)])


def _run_deg(bdst, counts):
    ones = jnp.ones((K, 8), jnp.float32)
    zeros = jnp.zeros((64, 8), jnp.float32)
    f = pl.kernel(
        _deg_body,
        out_type=jax.ShapeDtypeStruct((NPAD, 8), jnp.float32),
        mesh=_mesh(),
        scratch_types=[
            pltpu.VMEM((K,), jnp.int32),
            pltpu.VMEM((16,), jnp.int32),
            pltpu.VMEM((K, 8), jnp.float32),
            pltpu.VMEM((16, 8), jnp.float32),
            pltpu.VMEM((64, 8), jnp.float32),
            pltpu.VMEM((16,), jnp.int32),
            pltpu.VMEM_SHARED((CHUNK + 8, 8), jnp.float32),
            pltpu.SemaphoreType.DMA,
        ],
    )
    return f(bdst, counts, ones, zeros)


R = 1024
GRID = (N + R - 1) // R


def _tcA_body(deg_ref, x_ref, w_ref, dinv_ref, y_ref):
    deg = deg_ref[:, 0] + 1.0
    dinv = lax.rsqrt(deg)
    dinv_ref[...] = dinv
    y_ref[...] = jnp.dot(x_ref[...], w_ref[...],
                         preferred_element_type=jnp.float32) * dinv[:, None]


def _run_tcA(deg8, x, W0):
    return pl.pallas_call(
        _tcA_body,
        grid=(GRID,),
        in_specs=[
            pl.BlockSpec((R, 8), lambda i: (i, 0)),
            pl.BlockSpec((R, D), lambda i: (i, 0)),
            pl.BlockSpec((D, D), lambda i: (0, 0)),
        ],
        out_specs=[
            pl.BlockSpec((R,), lambda i: (i,)),
            pl.BlockSpec((R, D), lambda i: (i, 0)),
        ],
        out_shape=[
            jax.ShapeDtypeStruct((N,), jnp.float32),
            jax.ShapeDtypeStruct((N, D), jnp.float32),
        ],
    )(deg8, x, W0)


def _tcB_body(s_ref, y_ref, x_ref, dinv_ref, b_ref, w_ref, xo_ref, yo_ref):
    dinv = dinv_ref[...]
    h = dinv[:, None] * (s_ref[...] + y_ref[...]) + b_ref[...]
    xn = jnp.maximum(h, 0.0) + x_ref[...]
    xo_ref[...] = xn
    yo_ref[...] = jnp.dot(xn, w_ref[...],
                          preferred_element_type=jnp.float32) * dinv[:, None]


def _run_tcB(S, y, x, dinv, b, W):
    return pl.pallas_call(
        _tcB_body,
        grid=(GRID,),
        in_specs=[
            pl.BlockSpec((R, D), lambda i: (i, 0)),
            pl.BlockSpec((R, D), lambda i: (i, 0)),
            pl.BlockSpec((R, D), lambda i: (i, 0)),
            pl.BlockSpec((R,), lambda i: (i,)),
            pl.BlockSpec((1, D), lambda i: (0, 0)),
            pl.BlockSpec((D, D), lambda i: (0, 0)),
        ],
        out_specs=[
            pl.BlockSpec((R, D), lambda i: (i, 0)),
            pl.BlockSpec((R, D), lambda i: (i, 0)),
        ],
        out_shape=[
            jax.ShapeDtypeStruct((N, D), jnp.float32),
            jax.ShapeDtypeStruct((N, D), jnp.float32),
        ],
    )(S, y, x, dinv, b, W)


def _tcC_body(s_ref, y_ref, x_ref, dinv_ref, b_ref, x0_ref, wa_ref, wb_ref,
              yf_ref):
    dinv = dinv_ref[...]
    h = dinv[:, None] * (s_ref[...] + y_ref[...]) + b_ref[...]
    x5 = jnp.maximum(h, 0.0) + x_ref[...]
    yf = (jnp.dot(x5, wa_ref[...], preferred_element_type=jnp.float32)
          + jnp.dot(x0_ref[...], wb_ref[...],
                    preferred_element_type=jnp.float32))
    yf_ref[...] = yf * dinv[:, None]


def _run_tcC(S, y, x, dinv, b, x0, Wfa, Wfb):
    return pl.pallas_call(
        _tcC_body,
        grid=(GRID,),
        in_specs=[
            pl.BlockSpec((R, D), lambda i: (i, 0)),
            pl.BlockSpec((R, D), lambda i: (i, 0)),
            pl.BlockSpec((R, D), lambda i: (i, 0)),
            pl.BlockSpec((R,), lambda i: (i,)),
            pl.BlockSpec((1, D), lambda i: (0, 0)),
            pl.BlockSpec((R, D), lambda i: (i, 0)),
            pl.BlockSpec((D, OUT), lambda i: (0, 0)),
            pl.BlockSpec((D, OUT), lambda i: (0, 0)),
        ],
        out_specs=pl.BlockSpec((R, OUT), lambda i: (i, 0)),
        out_shape=jax.ShapeDtypeStruct((N, OUT), jnp.float32),
    )(S, y, x, dinv, b, x0, Wfa, Wfb)


def _tcD_body(s_ref, y_ref, dinv_ref, b_ref, x0_ref, m_ref, o_ref):
    dinv = dinv_ref[...]
    g = dinv[:, None] * (s_ref[...] + y_ref[...]) + b_ref[...]
    g = jax.nn.sigmoid(g) * 255.0
    x0 = x0_ref[...]
    mean = x0[:, 0:8]
    for k in range(1, 12):
        mean = mean + x0[:, 8 * k:8 * (k + 1)]
    mean = mean * (1.0 / 12.0)
    t = jnp.concatenate([mean] * 6, axis=1)
    o_ref[...] = (g + t) * m_ref[...][:, None]


def _run_tcD(S, yf, dinv, bf, x0, mask):
    return pl.pallas_call(
        _tcD_body,
        grid=(GRID,),
        in_specs=[
            pl.BlockSpec((R, OUT), lambda i: (i, 0)),
            pl.BlockSpec((R, OUT), lambda i: (i, 0)),
            pl.BlockSpec((R,), lambda i: (i,)),
            pl.BlockSpec((1, OUT), lambda i: (0, 0)),
            pl.BlockSpec((R, D), lambda i: (i, 0)),
            pl.BlockSpec((R,), lambda i: (i,)),
        ],
        out_specs=pl.BlockSpec((R, OUT), lambda i: (i, 0)),
        out_shape=jax.ShapeDtypeStruct((N, OUT), jnp.float32),
    )(S, yf, dinv, bf, x0, mask)


def kernel(x, edge_index, mask, W0, b0, W1, b1, W2, b2, W3, b3, W4, b4,
           Wf, bf):
    Ws = [W0, W1, W2, W3, W4]
    bs = [jnp.reshape(b, (1, D)) for b in (b0, b1, b2, b3, b4)]
    pad = jnp.zeros((EPAD - E,), jnp.int32)
    srcp = jnp.concatenate([edge_index[0], pad])
    dstp = jnp.concatenate([edge_index[1], pad])

    bsrc, bdst, counts = _run_partition(srcp, dstp)
    deg8 = _run_deg(bdst, counts)

    x0 = x
    dinv, y = _run_tcA(deg8, x, Ws[0])
    for i in range(4):
        S = _run_scatter(y, bsrc, bdst, counts, D)
        x, y = _run_tcB(S, y, x, dinv, bs[i], Ws[i + 1])
    S = _run_scatter(y, bsrc, bdst, counts, D)
    yf = _run_tcC(S, y, x, dinv, bs[4], x0, Wf[:D], Wf[D:])
    Sf = _run_scatter(yf, bsrc, bdst, counts, OUT)
    out = _run_tcD(Sf, yf, dinv, jnp.reshape(bf, (1, OUT)), x0, mask)
    return out[None]


# speedup vs baseline: 7.2949x; 7.2949x over previous
import functools

import jax
import jax.numpy as jnp
from jax import lax
from jax.experimental import pallas as pl
from jax.experimental.pallas import tpu as pltpu
from jax.experimental.pallas import tpu_sc as plsc

N = 215820
E = 647460
D = 96
OUT = 48

NC = 2
NS = 16
NW = NC * NS

CSH = 14
CHUNK = 1 << CSH
C = (N + CHUNK - 1) // CHUNK
NPAD = C * CHUNK

EPT = 20240
EPAD = EPT * NW

BLK = 4048
NV = BLK // 16
NBLKS = EPT // BLK

FB = 2048
FTH = 2016
CAP = 22400

K = 128

_mesh = functools.partial(
    plsc.VectorSubcoreMesh, core_axis_name="c", subcore_axis_name="s",
    num_cores=NC, num_subcores=NS)


def _wid():
    return lax.axis_index("s") * NC + lax.axis_index("c")


def _partition_body(srcp, dstp, bsrc, bdst, counts,
                    sbuf, dbuf, fsrc, fdst, cnt_v, ptrs, outc):
    wid = _wid()
    base = wid * EPT
    for c in range(C):
        ptrs[c] = jnp.int32(0)
        outc[c] = jnp.int32(0)

    for blk in range(NBLKS):
        gbase = base + blk * BLK
        pltpu.sync_copy(srcp.at[pl.ds(pl.multiple_of(gbase, 16), BLK)], sbuf)
        pltpu.sync_copy(dstp.at[pl.ds(pl.multiple_of(gbase, 16), BLK)], dbuf)

        def vbody(v, _, gbase=gbase):
            sv = sbuf[pl.ds(v * 16, 16)]
            dv = dbuf[pl.ds(v * 16, 16)]
            valid = (gbase + v * 16 + lax.iota(jnp.int32, 16)) < E
            ch = lax.shift_right_logical(dv, CSH)
            dl = jnp.bitwise_and(dv, CHUNK - 1)
            for c in range(C):
                m = jnp.logical_and(ch == c, valid)
                p = ptrs[c]
                mint = m.astype(jnp.int32)
                cums = plsc.cumsum(mint)
                pos = jnp.where(m, c * FB + p + cums - mint,
                                jnp.int32(C * FB - 1))
                plsc.store_scatter(fsrc, [pos], sv)
                plsc.store_scatter(fdst, [pos], dl)
                cnt = jnp.max(cums)
                p2 = p + cnt
                ptrs[c] = p2

                @pl.when(p2 >= FTH)
                def _(c=c, p2=p2):
                    oc = outc[c]
                    pltpu.sync_copy(fsrc.at[pl.ds(c * FB, FTH)],
                                    bsrc.at[pl.ds(pl.multiple_of((wid * C + c) * CAP + oc, 16), FTH)])
                    pltpu.sync_copy(fdst.at[pl.ds(c * FB, FTH)],
                                    bdst.at[pl.ds(pl.multiple_of((wid * C + c) * CAP + oc, 16), FTH)])
                    outc[c] = oc + FTH
                    fsrc[pl.ds(c * FB, 16)] = fsrc[pl.ds(c * FB + FTH, 16)]
                    fdst[pl.ds(c * FB, 16)] = fdst[pl.ds(c * FB + FTH, 16)]
                    ptrs[c] = p2 - FTH
            return 0

        lax.fori_loop(0, NV, vbody, 0)

    lanes = lax.iota(jnp.int32, 16)
    cvals = jnp.zeros((16,), jnp.int32)
    for c in range(C):
        p = ptrs[c]
        fsrc[pl.ds(c * FB + p, 16)] = jnp.zeros((16,), jnp.int32)
        fdst[pl.ds(c * FB + p, 16)] = jnp.full((16,), CHUNK, jnp.int32)
        p16 = jnp.bitwise_and(p + 15, jnp.int32(~15))
        oc = outc[c]
        pltpu.sync_copy(fsrc.at[pl.ds(c * FB, FTH)],
                        bsrc.at[pl.ds(pl.multiple_of((wid * C + c) * CAP + oc, 16), FTH)])
        pltpu.sync_copy(fdst.at[pl.ds(c * FB, FTH)],
                        bdst.at[pl.ds(pl.multiple_of((wid * C + c) * CAP + oc, 16), FTH)])
        cvals = jnp.where(lanes == c, oc + p16, cvals)
    cnt_v[...] = cvals
    pltpu.sync_copy(cnt_v, counts.at[pl.ds(pl.multiple_of(wid * 16, 16), 16)])


def _run_partition(srcp, dstp):
    f = pl.kernel(
        _partition_body,
        out_type=[
            jax.ShapeDtypeStruct((NW * C * CAP,), jnp.int32),
            jax.ShapeDtypeStruct((NW * C * CAP,), jnp.int32),
            jax.ShapeDtypeStruct((NW * 16,), jnp.int32),
        ],
        mesh=_mesh(),
        compiler_params=pltpu.CompilerParams(needs_layout_passes=False),
        scratch_types=[
            pltpu.VMEM((BLK,), jnp.int32),
            pltpu.VMEM((BLK,), jnp.int32),
            pltpu.VMEM((C * FB,), jnp.int32),
            pltpu.VMEM((C * FB,), jnp.int32),
            pltpu.VMEM((16,), jnp.int32),
            pltpu.SMEM((16,), jnp.int32),
            pltpu.SMEM((16,), jnp.int32),
        ],
    )
    return f(srcp, dstp)


def _scatter_body(y_hbm, bsrc, bdst, counts, zeros_hbm, S_hbm,
                  ids_s, ids_d, ids_s16, ids_d16, rows, rows16,
                  zbuf, counts_v, acc, sem):
    core = lax.axis_index("c")
    sid = lax.axis_index("s")
    tb = sid * (CHUNK // NS)
    pltpu.sync_copy(counts.at[pl.ds(pl.multiple_of(sid * NC * 16, 16), NC * 16)],
                    counts_v)
    pltpu.sync_copy(zeros_hbm, zbuf)
    cvecs = [counts_v[pl.ds(16 * h, 16)] for h in range(NC)]

    for c in range(C):
        @pl.when(jnp.equal(core, c % NC))
        def _(c=c):
            for z in range(CHUNK // NS // 64):
                pltpu.sync_copy(zbuf, acc.at[pl.ds(tb + z * 64, 64)])
            plsc.subcore_barrier()

            for h in range(NC):
                w2 = sid * NC + h
                cnt = cvecs[h][c]
                nblk = lax.shift_right_logical(cnt, 7)

                def blkbody(b, _, w2=w2, c=c):
                    off = b * K
                    pltpu.sync_copy(bsrc.at[pl.ds(pl.multiple_of((w2 * C + c) * CAP + off, 16), K)], ids_s)
                    pltpu.sync_copy(bdst.at[pl.ds(pl.multiple_of((w2 * C + c) * CAP + off, 16), K)], ids_d)
                    pltpu.async_copy(y_hbm.at[ids_s], rows, sem).wait()
                    pltpu.sync_copy(rows, acc.at[ids_d], add=True)
                    return 0

                lax.fori_loop(0, nblk, blkbody, 0)

                rem16 = jnp.bitwise_and(lax.shift_right_logical(cnt, 4), 7)
                tail0 = nblk * K

                def tbody(b, _, w2=w2, c=c, tail0=tail0):
                    off = tail0 + b * 16
                    pltpu.sync_copy(bsrc.at[pl.ds(pl.multiple_of((w2 * C + c) * CAP + off, 16), 16)], ids_s16)
                    pltpu.sync_copy(bdst.at[pl.ds(pl.multiple_of((w2 * C + c) * CAP + off, 16), 16)], ids_d16)
                    pltpu.async_copy(y_hbm.at[ids_s16], rows16, sem).wait()
                    pltpu.sync_copy(rows16, acc.at[ids_d16], add=True)
                    return 0

                lax.fori_loop(0, rem16, tbody, 0)
            plsc.subcore_barrier()
            for w in range(CHUNK // NS // K):
                pltpu.sync_copy(acc.at[pl.ds(tb + w * K, K)], rows)
                pltpu.sync_copy(rows,
                                S_hbm.at[pl.ds(c * CHUNK + tb + w * K, K)])


def _make_scatter(width):
    return pl.kernel(
        _scatter_body,
        out_type=jax.ShapeDtypeStruct((NPAD, width), jnp.float32),
        mesh=_mesh(),
        compiler_params=pltpu.CompilerParams(
            needs_layout_passes=False, use_tc_tiling_on_sc=False),
        scratch_types=[
            pltpu.VMEM((K,), jnp.int32),
            pltpu.VMEM((K,), jnp.int32),
            pltpu.VMEM((16,), jnp.int32),
            pltpu.VMEM((16,), jnp.int32),
            pltpu.VMEM((K, width), jnp.float32),
            pltpu.VMEM((16, width), jnp.float32),
            pltpu.VMEM((64, width), jnp.float32),
            pltpu.VMEM((NC * 16,), jnp.int32),
            pltpu.VMEM_SHARED((CHUNK + 8, width), jnp.float32),
            pltpu.SemaphoreType.DMA,
        ],
    )


def _run_scatter(y, bsrc, bdst, counts, width):
    zeros = jnp.zeros((64, width), jnp.float32)
    return _make_scatter(width)(y, bsrc, bdst, counts, zeros)


def _deg_body(bdst, counts, ones_hbm, zeros_hbm, deg_hbm,
              ids_d, ids_d16, onesv, ones16, zbuf, wb8, counts_v, acc, sem):
    core = lax.axis_index("c")
    sid = lax.axis_index("s")
    tb = sid * (CHUNK // NS)
    pltpu.sync_copy(counts.at[pl.ds(pl.multiple_of(sid * NC * 16, 16), NC * 16)],
                    counts_v)
    pltpu.sync_copy(ones_hbm, onesv)
    pltpu.sync_copy(ones_hbm.at[pl.ds(0, 16)], ones16)
    pltpu.sync_copy(zeros_hbm, zbuf)
    cvecs = [counts_v[pl.ds(16 * h, 16)] for h in range(NC)]

    for c in range(C):
        @pl.when(jnp.equal(core, c % NC))
        def _(c=c):
            for z in range(CHUNK // NS // 64):
                pltpu.sync_copy(zbuf, acc.at[pl.ds(tb + z * 64, 64)])
            plsc.subcore_barrier()

            for h in range(NC):
                w2 = sid * NC + h
                cnt = cvecs[h][c]
                nblk = lax.shift_right_logical(cnt, 7)

                def blkbody(b, _, w2=w2, c=c):
                    off = b * K
                    pltpu.sync_copy(bdst.at[pl.ds(pl.multiple_of((w2 * C + c) * CAP + off, 16), K)], ids_d)
                    pltpu.sync_copy(onesv, acc.at[ids_d], add=True)
                    return 0

                lax.fori_loop(0, nblk, blkbody, 0)

                rem16 = jnp.bitwise_and(lax.shift_right_logical(cnt, 4), 7)
                tail0 = nblk * K

                def tbody(b, _, w2=w2, c=c, tail0=tail0):
                    off = tail0 + b * 16
                    pltpu.sync_copy(bdst.at[pl.ds(pl.multiple_of((w2 * C + c) * CAP + off, 16), 16)], ids_d16)
                    pltpu.sync_copy(ones16, acc.at[ids_d16], add=True)
                    return 0

                lax.fori_loop(0, rem16, tbody, 0)
            plsc.subcore_barrier()
            for w in range(CHUNK // NS // K):
                pltpu.sync_copy(acc.at[pl.ds(tb + w * K, K)], wb8)
                pltpu.sync_copy(wb8,
                                deg_hbm.at[pl.ds(c * CHUNK + tb + w * K, K)])


def _run_deg(bdst, counts):
    ones = jnp.ones((K, 8), jnp.float32)
    zeros = jnp.zeros((64, 8), jnp.float32)
    f = pl.kernel(
        _deg_body,
        out_type=jax.ShapeDtypeStruct((NPAD, 8), jnp.float32),
        mesh=_mesh(),
        compiler_params=pltpu.CompilerParams(
            needs_layout_passes=False, use_tc_tiling_on_sc=False),
        scratch_types=[
            pltpu.VMEM((K,), jnp.int32),
            pltpu.VMEM((16,), jnp.int32),
            pltpu.VMEM((K, 8), jnp.float32),
            pltpu.VMEM((16, 8), jnp.float32),
            pltpu.VMEM((64, 8), jnp.float32),
            pltpu.VMEM((K, 8), jnp.float32),
            pltpu.VMEM((NC * 16,), jnp.int32),
            pltpu.VMEM_SHARED((CHUNK + 8, 8), jnp.float32),
            pltpu.SemaphoreType.DMA,
        ],
    )
    return f(bdst, counts, ones, zeros)


R = 1024
GRID = (N + R - 1) // R


def _tcA_body(deg_ref, x_ref, w_ref, dinv_ref, y_ref):
    deg = deg_ref[:, 0] + 1.0
    dinv = lax.rsqrt(deg)
    dinv_ref[...] = dinv
    y_ref[...] = jnp.dot(x_ref[...], w_ref[...],
                         preferred_element_type=jnp.float32) * dinv[:, None]


def _run_tcA(deg8, x, W0):
    return pl.pallas_call(
        _tcA_body,
        grid=(GRID,),
        in_specs=[
            pl.BlockSpec((R, 8), lambda i: (i, 0)),
            pl.BlockSpec((R, D), lambda i: (i, 0)),
            pl.BlockSpec((D, D), lambda i: (0, 0)),
        ],
        out_specs=[
            pl.BlockSpec((R,), lambda i: (i,)),
            pl.BlockSpec((R, D), lambda i: (i, 0)),
        ],
        out_shape=[
            jax.ShapeDtypeStruct((N,), jnp.float32),
            jax.ShapeDtypeStruct((N, D), jnp.float32),
        ],
    )(deg8, x, W0)


def _tcB_body(s_ref, y_ref, x_ref, dinv_ref, b_ref, w_ref, xo_ref, yo_ref):
    dinv = dinv_ref[...]
    h = dinv[:, None] * (s_ref[...] + y_ref[...]) + b_ref[...]
    xn = jnp.maximum(h, 0.0) + x_ref[...]
    xo_ref[...] = xn
    yo_ref[...] = jnp.dot(xn, w_ref[...],
                          preferred_element_type=jnp.float32) * dinv[:, None]


def _run_tcB(S, y, x, dinv, b, W):
    return pl.pallas_call(
        _tcB_body,
        grid=(GRID,),
        in_specs=[
            pl.BlockSpec((R, D), lambda i: (i, 0)),
            pl.BlockSpec((R, D), lambda i: (i, 0)),
            pl.BlockSpec((R, D), lambda i: (i, 0)),
            pl.BlockSpec((R,), lambda i: (i,)),
            pl.BlockSpec((1, D), lambda i: (0, 0)),
            pl.BlockSpec((D, D), lambda i: (0, 0)),
        ],
        out_specs=[
            pl.BlockSpec((R, D), lambda i: (i, 0)),
            pl.BlockSpec((R, D), lambda i: (i, 0)),
        ],
        out_shape=[
            jax.ShapeDtypeStruct((N, D), jnp.float32),
            jax.ShapeDtypeStruct((N, D), jnp.float32),
        ],
    )(S, y, x, dinv, b, W)


def _tcC_body(s_ref, y_ref, x_ref, dinv_ref, b_ref, x0_ref, wa_ref, wb_ref,
              yf_ref):
    dinv = dinv_ref[...]
    h = dinv[:, None] * (s_ref[...] + y_ref[...]) + b_ref[...]
    x5 = jnp.maximum(h, 0.0) + x_ref[...]
    yf = (jnp.dot(x5, wa_ref[...], preferred_element_type=jnp.float32)
          + jnp.dot(x0_ref[...], wb_ref[...],
                    preferred_element_type=jnp.float32))
    yf_ref[...] = yf * dinv[:, None]


def _run_tcC(S, y, x, dinv, b, x0, Wfa, Wfb):
    return pl.pallas_call(
        _tcC_body,
        grid=(GRID,),
        in_specs=[
            pl.BlockSpec((R, D), lambda i: (i, 0)),
            pl.BlockSpec((R, D), lambda i: (i, 0)),
            pl.BlockSpec((R, D), lambda i: (i, 0)),
            pl.BlockSpec((R,), lambda i: (i,)),
            pl.BlockSpec((1, D), lambda i: (0, 0)),
            pl.BlockSpec((R, D), lambda i: (i, 0)),
            pl.BlockSpec((D, OUT), lambda i: (0, 0)),
            pl.BlockSpec((D, OUT), lambda i: (0, 0)),
        ],
        out_specs=pl.BlockSpec((R, OUT), lambda i: (i, 0)),
        out_shape=jax.ShapeDtypeStruct((N, OUT), jnp.float32),
    )(S, y, x, dinv, b, x0, Wfa, Wfb)


def _tcD_body(s_ref, y_ref, dinv_ref, b_ref, x0_ref, m_ref, o_ref):
    dinv = dinv_ref[...]
    g = dinv[:, None] * (s_ref[...] + y_ref[...]) + b_ref[...]
    g = jax.nn.sigmoid(g) * 255.0
    x0 = x0_ref[...]
    mean = x0[:, 0:8]
    for k in range(1, 12):
        mean = mean + x0[:, 8 * k:8 * (k + 1)]
    mean = mean * (1.0 / 12.0)
    t = jnp.concatenate([mean] * 6, axis=1)
    o_ref[...] = (g + t) * m_ref[...][:, None]


def _run_tcD(S, yf, dinv, bf, x0, mask):
    return pl.pallas_call(
        _tcD_body,
        grid=(GRID,),
        in_specs=[
            pl.BlockSpec((R, OUT), lambda i: (i, 0)),
            pl.BlockSpec((R, OUT), lambda i: (i, 0)),
            pl.BlockSpec((R,), lambda i: (i,)),
            pl.BlockSpec((1, OUT), lambda i: (0, 0)),
            pl.BlockSpec((R, D), lambda i: (i, 0)),
            pl.BlockSpec((R,), lambda i: (i,)),
        ],
        out_specs=pl.BlockSpec((R, OUT), lambda i: (i, 0)),
        out_shape=jax.ShapeDtypeStruct((N, OUT), jnp.float32),
    )(S, yf, dinv, bf, x0, mask)


def kernel(x, edge_index, mask, W0, b0, W1, b1, W2, b2, W3, b3, W4, b4,
           Wf, bf):
    Ws = [W0, W1, W2, W3, W4]
    bs = [jnp.reshape(b, (1, D)) for b in (b0, b1, b2, b3, b4)]
    pad = jnp.zeros((EPAD - E,), jnp.int32)
    srcp = jnp.concatenate([edge_index[0], pad])
    dstp = jnp.concatenate([edge_index[1], pad])

    bsrc, bdst, counts = _run_partition(srcp, dstp)
    deg8 = _run_deg(bdst, counts)

    x0 = x
    dinv, y = _run_tcA(deg8, x, Ws[0])
    for i in range(4):
        S = _run_scatter(y, bsrc, bdst, counts, D)
        x, y = _run_tcB(S, y, x, dinv, bs[i], Ws[i + 1])
    S = _run_scatter(y, bsrc, bdst, counts, D)
    yf = _run_tcC(S, y, x, dinv, bs[4], x0, Wf[:D], Wf[D:])
    Sf = _run_scatter(yf, bsrc, bdst, counts, OUT)
    out = _run_tcD(Sf, yf, dinv, jnp.reshape(bf, (1, OUT)), x0, mask)
    return out[None]


# speedup vs baseline: 10.5890x; 1.4516x over previous
import functools

import jax
import jax.numpy as jnp
from jax import lax
from jax.experimental import pallas as pl
from jax.experimental.pallas import tpu as pltpu
from jax.experimental.pallas import tpu_sc as plsc

N = 215820
E = 647460
D = 96
OUT = 48

NC = 2
NS = 16
NW = NC * NS

CSH = 14
CHUNK = 1 << CSH
C = (N + CHUNK - 1) // CHUNK
NPAD = C * CHUNK

EPT = 20240
EPAD = EPT * NW

BLK = 4048
NV = BLK // 16
NBLKS = EPT // BLK

FB = 2048
FTH = 1920
CAP = 22400

K = 128
G = 2
CROWS = CAP // K

_mesh = functools.partial(
    plsc.VectorSubcoreMesh, core_axis_name="c", subcore_axis_name="s",
    num_cores=NC, num_subcores=NS)


def _wid():
    return lax.axis_index("s") * NC + lax.axis_index("c")


def _partition_body(srcp, dstp, bsrc, bdst, counts,
                    sbuf, dbuf, fsrc, fdst, cnt_v, ptrs, outc):
    wid = _wid()
    base = wid * EPT
    for c in range(C):
        ptrs[c] = jnp.int32(0)
        outc[c] = jnp.int32(0)

    for blk in range(NBLKS):
        gbase = base + blk * BLK
        pltpu.sync_copy(srcp.at[pl.ds(pl.multiple_of(gbase, 16), BLK)], sbuf)
        pltpu.sync_copy(dstp.at[pl.ds(pl.multiple_of(gbase, 16), BLK)], dbuf)

        def vbody(v, _, gbase=gbase):
            sv = sbuf[pl.ds(v * 16, 16)]
            dv = dbuf[pl.ds(v * 16, 16)]
            valid = (gbase + v * 16 + lax.iota(jnp.int32, 16)) < E
            ch = lax.shift_right_logical(dv, CSH)
            dl = jnp.bitwise_and(dv, CHUNK - 1)
            for c in range(C):
                m = jnp.logical_and(ch == c, valid)
                p = ptrs[c]
                mint = m.astype(jnp.int32)
                cums = plsc.cumsum(mint)
                pos = jnp.where(m, c * FB + p + cums - mint,
                                jnp.int32(C * FB - 1))
                plsc.store_scatter(fsrc, [pos], sv)
                plsc.store_scatter(fdst, [pos], dl)
                cnt = jnp.max(cums)
                p2 = p + cnt
                ptrs[c] = p2

                @pl.when(p2 >= FTH)
                def _(c=c, p2=p2):
                    oc = outc[c]
                    pltpu.sync_copy(fsrc.at[pl.ds(c * FB, FTH)],
                                    bsrc.at[pl.ds(pl.multiple_of((wid * C + c) * CAP + oc, 16), FTH)])
                    pltpu.sync_copy(fdst.at[pl.ds(c * FB, FTH)],
                                    bdst.at[pl.ds(pl.multiple_of((wid * C + c) * CAP + oc, 16), FTH)])
                    outc[c] = oc + FTH
                    fsrc[pl.ds(c * FB, 16)] = fsrc[pl.ds(c * FB + FTH, 16)]
                    fdst[pl.ds(c * FB, 16)] = fdst[pl.ds(c * FB + FTH, 16)]
                    ptrs[c] = p2 - FTH
            return 0

        lax.fori_loop(0, NV, vbody, 0)

    lanes = lax.iota(jnp.int32, 16)
    cvals = jnp.zeros((16,), jnp.int32)
    sent_d = CHUNK + jnp.bitwise_and(lanes, 7)
    for c in range(C):
        p = ptrs[c]
        sent_s = lanes * 512 + c * 32
        for j in range(8):
            fsrc[pl.ds(c * FB + p + 16 * j, 16)] = sent_s + 16 * j
            fdst[pl.ds(c * FB + p + 16 * j, 16)] = sent_d
        p128 = jnp.bitwise_and(p + 127, jnp.int32(~127))
        oc = outc[c]
        pltpu.sync_copy(fsrc.at[pl.ds(c * FB, FB)],
                        bsrc.at[pl.ds(pl.multiple_of((wid * C + c) * CAP + oc, 16), FB)])
        pltpu.sync_copy(fdst.at[pl.ds(c * FB, FB)],
                        bdst.at[pl.ds(pl.multiple_of((wid * C + c) * CAP + oc, 16), FB)])
        cvals = jnp.where(lanes == c, oc + p128, cvals)
    cnt_v[...] = cvals
    pltpu.sync_copy(cnt_v, counts.at[pl.ds(pl.multiple_of(wid * 16, 16), 16)])


def _run_partition(srcp, dstp):
    f = pl.kernel(
        _partition_body,
        out_type=[
            jax.ShapeDtypeStruct((NW * C * CAP,), jnp.int32),
            jax.ShapeDtypeStruct((NW * C * CAP,), jnp.int32),
            jax.ShapeDtypeStruct((NW * 16,), jnp.int32),
        ],
        mesh=_mesh(),
        compiler_params=pltpu.CompilerParams(needs_layout_passes=False),
        scratch_types=[
            pltpu.VMEM((BLK,), jnp.int32),
            pltpu.VMEM((BLK,), jnp.int32),
            pltpu.VMEM((C * FB,), jnp.int32),
            pltpu.VMEM((C * FB,), jnp.int32),
            pltpu.VMEM((16,), jnp.int32),
            pltpu.SMEM((16,), jnp.int32),
            pltpu.SMEM((16,), jnp.int32),
        ],
    )
    return f(srcp, dstp)


def _vextract(vec, idx):
    return jnp.max(jnp.where(lax.iota(jnp.int32, 16) == idx, vec, 0))


def _scatter_body(y_hbm, bsrc, bdst, counts, zeros_hbm, S_hbm,
                  is0, is1, id0, id1, r0, r1,
                  zbuf, counts_v, acc, gsems, ssems, zsem):
    ids_s = [is0, is1]
    ids_d = [id0, id1]
    rows = [r0, r1]
    core = lax.axis_index("c")
    sid = lax.axis_index("s")
    tb = sid * (CHUNK // NS)
    pltpu.sync_copy(counts.at[pl.ds(pl.multiple_of(sid * NC * 16, 16), NC * 16)],
                    counts_v)
    pltpu.sync_copy(zeros_hbm, zbuf)
    cvecs = [counts_v[pl.ds(16 * h, 16)] for h in range(NC)]

    def cbody(cq, _):
        c = cq * NC + core
        zd = [pltpu.async_copy(zbuf, acc.at[pl.ds(tb + z * 64, 64)], zsem)
              for z in range(CHUNK // NS // 64)]
        for d in zd:
            d.wait()
        plsc.subcore_barrier()

        for h in range(NC):
            w2 = sid * NC + h
            cnt = _vextract(cvecs[h], c)
            nblk = lax.shift_right_logical(cnt, 7)
            base0 = (w2 * C + c) * CAP

            def rbody(r, _, base0=base0, nblk=nblk):
                blk0 = r * G
                for g in range(G):
                    blk = blk0 + g

                    @pl.when(jnp.logical_and(blk < nblk, blk >= G))
                    def _(g=g):
                        pltpu.make_async_copy(rows[g], acc.at[pl.ds(0, K)],
                                              ssems[g]).wait()

                    @pl.when(blk < nblk)
                    def _(g=g, blk=blk):
                        off = pl.multiple_of(base0 + blk * K, 16)
                        pltpu.sync_copy(bsrc.at[pl.ds(off, K)], ids_s[g])
                        pltpu.sync_copy(bdst.at[pl.ds(off, K)], ids_d[g])
                        pltpu.async_copy(y_hbm.at[ids_s[g]], rows[g],
                                         gsems[g])
                for g in range(G):
                    blk = blk0 + g

                    @pl.when(blk < nblk)
                    def _(g=g):
                        pltpu.make_async_copy(y_hbm.at[ids_s[g]], rows[g],
                                              gsems[g]).wait()
                        pltpu.async_copy(rows[g], acc.at[ids_d[g]],
                                         ssems[g], add=True)
                return 0

            nrnd = lax.div(nblk + G - 1, jnp.int32(G))
            lax.fori_loop(0, nrnd, rbody, 0)
            for g in range(G):
                @pl.when(g < nblk)
                def _(g=g):
                    pltpu.make_async_copy(rows[g], acc.at[pl.ds(0, K)],
                                          ssems[g]).wait()
        plsc.subcore_barrier()
        nw_ = CHUNK // NS // K
        obase = pl.multiple_of(c * CHUNK + tb, 16)
        pltpu.sync_copy(acc.at[pl.ds(tb, K)], rows[0])
        for w in range(nw_):
            pltpu.async_copy(rows[w % 2],
                             S_hbm.at[pl.ds(obase + w * K, K)], zsem)
            if w + 1 < nw_:
                pltpu.sync_copy(acc.at[pl.ds(tb + (w + 1) * K, K)],
                                rows[(w + 1) % 2])
            pltpu.make_async_copy(rows[w % 2],
                                  S_hbm.at[pl.ds(obase + w * K, K)],
                                  zsem).wait()
        return 0

    lax.fori_loop(0, C // NC, cbody, 0)


def _make_scatter(width):
    return pl.kernel(
        _scatter_body,
        out_type=jax.ShapeDtypeStruct((NPAD, width), jnp.float32),
        mesh=_mesh(),
        compiler_params=pltpu.CompilerParams(
            needs_layout_passes=False, use_tc_tiling_on_sc=False),
        scratch_types=(
            [pltpu.VMEM((K,), jnp.int32)] * (2 * G)
            + [pltpu.VMEM((K, width), jnp.float32)] * G
            + [
                pltpu.VMEM((64, width), jnp.float32),
                pltpu.VMEM((NC * 16,), jnp.int32),
                pltpu.VMEM_SHARED((CHUNK + 8, width), jnp.float32),
                [pltpu.SemaphoreType.DMA] * G,
                [pltpu.SemaphoreType.DMA] * G,
                pltpu.SemaphoreType.DMA,
            ]
        ),
    )


def _run_scatter(y, bsrc, bdst, counts, width):
    zeros = jnp.zeros((64, width), jnp.float32)
    return _make_scatter(width)(y, bsrc, bdst, counts, zeros)


def _deg_body(bdst, counts, ones_hbm, zeros_hbm, deg_hbm,
              id0, id1,
              onesv, zbuf, wb8, counts_v, acc, asems, zsem):
    ids_d = [id0, id1]
    core = lax.axis_index("c")
    sid = lax.axis_index("s")
    tb = sid * (CHUNK // NS)
    pltpu.sync_copy(counts.at[pl.ds(pl.multiple_of(sid * NC * 16, 16), NC * 16)],
                    counts_v)
    pltpu.sync_copy(ones_hbm, onesv)
    pltpu.sync_copy(zeros_hbm, zbuf)
    cvecs = [counts_v[pl.ds(16 * h, 16)] for h in range(NC)]

    def cbody(cq, _):
        c = cq * NC + core
        zd = [pltpu.async_copy(zbuf, acc.at[pl.ds(tb + z * 64, 64)], zsem)
              for z in range(CHUNK // NS // 64)]
        for d in zd:
            d.wait()
        plsc.subcore_barrier()

        for h in range(NC):
            w2 = sid * NC + h
            cnt = _vextract(cvecs[h], c)
            nblk = lax.shift_right_logical(cnt, 7)
            base0 = (w2 * C + c) * CAP

            def rbody(r, _, base0=base0, nblk=nblk):
                blk0 = r * G
                for g in range(G):
                    blk = blk0 + g

                    @pl.when(jnp.logical_and(blk < nblk, blk >= G))
                    def _(g=g):
                        pltpu.make_async_copy(onesv, acc.at[pl.ds(0, K)],
                                              asems[g]).wait()

                    @pl.when(blk < nblk)
                    def _(g=g, blk=blk):
                        off = pl.multiple_of(base0 + blk * K, 16)
                        pltpu.sync_copy(bdst.at[pl.ds(off, K)], ids_d[g])
                        pltpu.async_copy(onesv, acc.at[ids_d[g]],
                                         asems[g], add=True)
                return 0

            nrnd = lax.div(nblk + G - 1, jnp.int32(G))
            lax.fori_loop(0, nrnd, rbody, 0)
            for g in range(G):
                @pl.when(g < nblk)
                def _(g=g):
                    pltpu.make_async_copy(onesv, acc.at[pl.ds(0, K)],
                                          asems[g]).wait()
        plsc.subcore_barrier()
        obase = pl.multiple_of(c * CHUNK + tb, 16)
        for w in range(CHUNK // NS // K):
            pltpu.sync_copy(acc.at[pl.ds(tb + w * K, K)], wb8)
            pltpu.sync_copy(wb8, deg_hbm.at[pl.ds(obase + w * K, K)])
        return 0

    lax.fori_loop(0, C // NC, cbody, 0)


def _run_deg(bdst, counts):
    ones = jnp.ones((K, 8), jnp.float32)
    zeros = jnp.zeros((64, 8), jnp.float32)
    f = pl.kernel(
        _deg_body,
        out_type=jax.ShapeDtypeStruct((NPAD, 8), jnp.float32),
        mesh=_mesh(),
        compiler_params=pltpu.CompilerParams(
            needs_layout_passes=False, use_tc_tiling_on_sc=False),
        scratch_types=(
            [pltpu.VMEM((K,), jnp.int32)] * G
            + [
                pltpu.VMEM((K, 8), jnp.float32),
                pltpu.VMEM((64, 8), jnp.float32),
                pltpu.VMEM((K, 8), jnp.float32),
                pltpu.VMEM((NC * 16,), jnp.int32),
                pltpu.VMEM_SHARED((CHUNK + 8, 8), jnp.float32),
                [pltpu.SemaphoreType.DMA] * G,
                pltpu.SemaphoreType.DMA,
            ]
        ),
    )
    return f(bdst, counts, ones, zeros)


R = 2048
GRID = (N + R - 1) // R


def _tcA_body(deg_ref, x_ref, w_ref, dinv_ref, y_ref):
    deg = deg_ref[:, 0] + 1.0
    dinv = lax.rsqrt(deg)
    dinv_ref[...] = dinv
    y_ref[...] = jnp.dot(x_ref[...], w_ref[...],
                         preferred_element_type=jnp.float32) * dinv[:, None]


def _run_tcA(deg8, x, W0):
    return pl.pallas_call(
        _tcA_body,
        grid=(GRID,),
        in_specs=[
            pl.BlockSpec((R, 8), lambda i: (i, 0)),
            pl.BlockSpec((R, D), lambda i: (i, 0)),
            pl.BlockSpec((D, D), lambda i: (0, 0)),
        ],
        out_specs=[
            pl.BlockSpec((R,), lambda i: (i,)),
            pl.BlockSpec((R, D), lambda i: (i, 0)),
        ],
        out_shape=[
            jax.ShapeDtypeStruct((N,), jnp.float32),
            jax.ShapeDtypeStruct((N, D), jnp.float32),
        ],
    )(deg8, x, W0)


def _tcB_body(s_ref, y_ref, x_ref, dinv_ref, b_ref, w_ref, xo_ref, yo_ref):
    dinv = dinv_ref[...]
    h = dinv[:, None] * (s_ref[...] + y_ref[...]) + b_ref[...]
    xn = jnp.maximum(h, 0.0) + x_ref[...]
    xo_ref[...] = xn
    yo_ref[...] = jnp.dot(xn, w_ref[...],
                          preferred_element_type=jnp.float32) * dinv[:, None]


def _run_tcB(S, y, x, dinv, b, W):
    return pl.pallas_call(
        _tcB_body,
        grid=(GRID,),
        in_specs=[
            pl.BlockSpec((R, D), lambda i: (i, 0)),
            pl.BlockSpec((R, D), lambda i: (i, 0)),
            pl.BlockSpec((R, D), lambda i: (i, 0)),
            pl.BlockSpec((R,), lambda i: (i,)),
            pl.BlockSpec((1, D), lambda i: (0, 0)),
            pl.BlockSpec((D, D), lambda i: (0, 0)),
        ],
        out_specs=[
            pl.BlockSpec((R, D), lambda i: (i, 0)),
            pl.BlockSpec((R, D), lambda i: (i, 0)),
        ],
        out_shape=[
            jax.ShapeDtypeStruct((N, D), jnp.float32),
            jax.ShapeDtypeStruct((N, D), jnp.float32),
        ],
    )(S, y, x, dinv, b, W)


def _tcC_body(s_ref, y_ref, x_ref, dinv_ref, b_ref, x0_ref, wa_ref, wb_ref,
              yf_ref):
    dinv = dinv_ref[...]
    h = dinv[:, None] * (s_ref[...] + y_ref[...]) + b_ref[...]
    x5 = jnp.maximum(h, 0.0) + x_ref[...]
    yf = (jnp.dot(x5, wa_ref[...], preferred_element_type=jnp.float32)
          + jnp.dot(x0_ref[...], wb_ref[...],
                    preferred_element_type=jnp.float32))
    yf_ref[...] = yf * dinv[:, None]


def _run_tcC(S, y, x, dinv, b, x0, Wfa, Wfb):
    return pl.pallas_call(
        _tcC_body,
        grid=(GRID,),
        in_specs=[
            pl.BlockSpec((R, D), lambda i: (i, 0)),
            pl.BlockSpec((R, D), lambda i: (i, 0)),
            pl.BlockSpec((R, D), lambda i: (i, 0)),
            pl.BlockSpec((R,), lambda i: (i,)),
            pl.BlockSpec((1, D), lambda i: (0, 0)),
            pl.BlockSpec((R, D), lambda i: (i, 0)),
            pl.BlockSpec((D, OUT), lambda i: (0, 0)),
            pl.BlockSpec((D, OUT), lambda i: (0, 0)),
        ],
        out_specs=pl.BlockSpec((R, OUT), lambda i: (i, 0)),
        out_shape=jax.ShapeDtypeStruct((N, OUT), jnp.float32),
    )(S, y, x, dinv, b, x0, Wfa, Wfb)


def _tcD_body(s_ref, y_ref, dinv_ref, b_ref, x0_ref, m_ref, t_ref, o_ref):
    dinv = dinv_ref[...]
    g = dinv[:, None] * (s_ref[...] + y_ref[...]) + b_ref[...]
    g = jax.nn.sigmoid(g) * 255.0
    tmean = jnp.dot(x0_ref[...], t_ref[...],
                    preferred_element_type=jnp.float32)
    o_ref[...] = (g + tmean) * m_ref[...][:, None]


def _run_tcD(S, yf, dinv, bf, x0, mask):
    tmat = jnp.tile(jnp.eye(8, dtype=jnp.float32) / 12.0, (12, 6))
    return pl.pallas_call(
        _tcD_body,
        grid=(GRID,),
        in_specs=[
            pl.BlockSpec((R, OUT), lambda i: (i, 0)),
            pl.BlockSpec((R, OUT), lambda i: (i, 0)),
            pl.BlockSpec((R,), lambda i: (i,)),
            pl.BlockSpec((1, OUT), lambda i: (0, 0)),
            pl.BlockSpec((R, D), lambda i: (i, 0)),
            pl.BlockSpec((R,), lambda i: (i,)),
            pl.BlockSpec((D, OUT), lambda i: (0, 0)),
        ],
        out_specs=pl.BlockSpec((R, OUT), lambda i: (i, 0)),
        out_shape=jax.ShapeDtypeStruct((N, OUT), jnp.float32),
    )(S, yf, dinv, bf, x0, mask, tmat)


def kernel(x, edge_index, mask, W0, b0, W1, b1, W2, b2, W3, b3, W4, b4,
           Wf, bf):
    Ws = [W0, W1, W2, W3, W4]
    bs = [jnp.reshape(b, (1, D)) for b in (b0, b1, b2, b3, b4)]
    pad = jnp.zeros((EPAD - E,), jnp.int32)
    srcp = jnp.concatenate([edge_index[0], pad])
    dstp = jnp.concatenate([edge_index[1], pad])

    bsrc, bdst, counts = _run_partition(srcp, dstp)
    deg8 = _run_deg(bdst, counts)

    x0 = x
    dinv, y = _run_tcA(deg8, x, Ws[0])
    for i in range(4):
        S = _run_scatter(y, bsrc, bdst, counts, D)
        x, y = _run_tcB(S, y, x, dinv, bs[i], Ws[i + 1])
    S = _run_scatter(y, bsrc, bdst, counts, D)
    yf = _run_tcC(S, y, x, dinv, bs[4], x0, Wf[:D], Wf[D:])
    Sf = _run_scatter(yf, bsrc, bdst, counts, OUT)
    out = _run_tcD(Sf, yf, dinv, jnp.reshape(bf, (1, OUT)), x0, mask)
    return out[None]
